# baseline (speedup 1.0000x reference)
import functools
import math

import jax
import jax.numpy as jnp
from jax import lax
from jax.experimental import pallas as pl
from jax.experimental.pallas import tpu as pltpu
from jax.experimental.pallas import tpu_sc as plsc

_NUM_HEADS = 16
_NUM_BUCKETS = 32
_MAX_DISTANCE = 128
_QL = 2048
_KL = 2048

_S = 512
_NSTRIP = _QL // _S
_MW = 4096

_THRESH = (12, 16, 23, 32, 46, 64, 91)

_SC_INFO = plsc.get_sparse_core_info()
_NWORK = _SC_INFO.num_cores * _SC_INFO.num_subcores
_CHUNK = _MW // _NWORK


def _sc_lookup_kernel(w_hbm, delta_hbm, vb_hbm, w_v, delta_v, out_v):
    wid = lax.axis_index("s") * _SC_INFO.num_cores + lax.axis_index("c")
    base = wid * _CHUNK
    pltpu.sync_copy(w_hbm, w_v)
    pltpu.sync_copy(delta_hbm, delta_v)
    delta = delta_v[...]
    nb = _NUM_BUCKETS // 2
    me = nb // 2
    dnums = lax.GatherDimensionNumbers(
        offset_dims=(), collapsed_slice_dims=(0,), start_index_map=(0,)
    )
    chunks = []
    for v in range(_CHUNK // 16):
        j = lax.iota(jnp.int32, 16) + (base + v * 16)
        d = j - 2047 + delta
        side = jnp.where(d > 0, nb, 0).astype(jnp.int32)
        a = jnp.abs(d)
        large = jnp.full((16,), me, jnp.int32)
        for t in _THRESH:
            large = large + jnp.where(a >= t, 1, 0).astype(jnp.int32)
        bucket = side + jnp.where(a < me, a, large)
        is_lo = bucket < 16
        lo_idx = jnp.where(is_lo, bucket, 0)
        hi_idx = jnp.where(is_lo, 0, bucket - 16)
        chunks.append((is_lo, lo_idx, hi_idx))
    for h in range(_NUM_HEADS):
        wlo = w_v[pl.ds(h * _NUM_BUCKETS, 16)]
        whi = w_v[pl.ds(h * _NUM_BUCKETS + 16, 16)]
        for v, (is_lo, lo_idx, hi_idx) in enumerate(chunks):
            glo = lax.gather(
                wlo, lo_idx[:, None], dnums, (1,),
                mode=lax.GatherScatterMode.PROMISE_IN_BOUNDS,
            )
            ghi = lax.gather(
                whi, hi_idx[:, None], dnums, (1,),
                mode=lax.GatherScatterMode.PROMISE_IN_BOUNDS,
            )
            out_v[h, pl.ds(v * 16, 16)] = jnp.where(is_lo, glo, ghi)
    pltpu.sync_copy(out_v, vb_hbm.at[:, pl.ds(base, _CHUNK)])


def _stream_kernel(vb_ref, out_ref, m_ref, sem_ref):
    h = pl.program_id(0)
    last = pl.num_programs(0) - 1
    buf = h % 2

    def strip_copies(b, hh):
        return [
            pltpu.make_async_copy(
                m_ref.at[b, :, pl.ds(_S * (_NSTRIP - 1 - i), _KL)],
                out_ref.at[0, hh, pl.ds(_S * i, _S), :],
                sem_ref.at[b],
            )
            for i in range(_NSTRIP)
        ]

    @pl.when(h >= 2)
    def _wait_prev():
        for c in strip_copies(buf, h - 2):
            c.wait()

    m_ref[buf] = pltpu.roll(
        jnp.broadcast_to(vb_ref[...].reshape(1, _MW), (_S, _MW)),
        _MW - (_S - 1),
        axis=1,
        stride=1,
        stride_axis=0,
    )

    for c in strip_copies(buf, h):
        c.start()

    @pl.when(h == last)
    def _drain():
        for c in strip_copies(buf, h):
            c.wait()
        for c in strip_copies(1 - buf, h - 1):
            c.wait()


@functools.partial(
    pl.kernel,
    out_type=jax.ShapeDtypeStruct((_NUM_HEADS, _MW), jnp.float32),
    mesh=plsc.VectorSubcoreMesh(core_axis_name="c", subcore_axis_name="s"),
    scratch_types=[
        pltpu.VMEM((_NUM_BUCKETS * _NUM_HEADS,), jnp.float32),
        pltpu.VMEM((16,), jnp.int32),
        pltpu.VMEM((_NUM_HEADS, _CHUNK), jnp.float32),
    ],
)
def _sc_lookup(w_hbm, delta_hbm, vb_hbm, w_v, delta_v, out_v):
    _sc_lookup_kernel(w_hbm, delta_hbm, vb_hbm, w_v, delta_v, out_v)


def kernel(W, q_len, k_len):
    delta = (
        jnp.asarray(k_len, jnp.int32) - _KL
        - (jnp.asarray(q_len, jnp.int32) - _QL)
    )
    vbase = _sc_lookup(
        W.T.reshape(_NUM_HEADS * _NUM_BUCKETS),
        jnp.full((16,), 1, jnp.int32) * delta,
    )
    vbase = vbase.reshape(_NUM_HEADS, 1, _MW)
    out = pl.pallas_call(
        _stream_kernel,
        grid=(_NUM_HEADS,),
        in_specs=[pl.BlockSpec((1, 1, _MW), lambda h: (h, 0, 0))],
        out_specs=pl.BlockSpec(memory_space=pl.ANY),
        out_shape=jax.ShapeDtypeStruct((1, _NUM_HEADS, _QL, _KL), jnp.float32),
        scratch_shapes=[
            pltpu.VMEM((2, _S, _MW), jnp.float32),
            pltpu.SemaphoreType.DMA((2,)),
        ],
    )(vbase)
    return out


# speedup vs baseline: 1.0203x; 1.0203x over previous
import functools

import jax
import jax.numpy as jnp
from jax import lax
from jax.experimental import pallas as pl
from jax.experimental.pallas import tpu as pltpu
from jax.experimental.pallas import tpu_sc as plsc

_NUM_HEADS = 16
_NUM_BUCKETS = 32
_MAX_DISTANCE = 128
_QL = 2048
_KL = 2048

_S = 512
_NSTRIP = _QL // _S
_MW = 4096

_THRESH = (12, 16, 23, 32, 46, 64, 91)

_SC_INFO = plsc.get_sparse_core_info()
_NWORK = _SC_INFO.num_cores * _SC_INFO.num_subcores
_CHUNK = _MW // _NWORK


def _sc_lookup_kernel(w_hbm, delta_hbm, vb_hbm, w_v, delta_v, out_v):
    wid = lax.axis_index("s") * _SC_INFO.num_cores + lax.axis_index("c")
    base = wid * _CHUNK
    pltpu.sync_copy(w_hbm, w_v)
    pltpu.sync_copy(delta_hbm, delta_v)
    delta = delta_v[...]
    nb = _NUM_BUCKETS // 2
    me = nb // 2
    dnums = lax.GatherDimensionNumbers(
        offset_dims=(), collapsed_slice_dims=(0,), start_index_map=(0,)
    )
    chunks = []
    for v in range(_CHUNK // 16):
        j = lax.iota(jnp.int32, 16) + (base + v * 16)
        d = j - 2047 + delta
        side = jnp.where(d > 0, nb, 0).astype(jnp.int32)
        a = jnp.abs(d)
        large = jnp.full((16,), me, jnp.int32)
        for t in _THRESH:
            large = large + jnp.where(a >= t, 1, 0).astype(jnp.int32)
        bucket = side + jnp.where(a < me, a, large)
        is_lo = bucket < 16
        lo_idx = jnp.where(is_lo, bucket, 0)
        hi_idx = jnp.where(is_lo, 0, bucket - 16)
        chunks.append((is_lo, lo_idx, hi_idx))
    for h in range(_NUM_HEADS):
        wlo = w_v[pl.ds(h * _NUM_BUCKETS, 16)]
        whi = w_v[pl.ds(h * _NUM_BUCKETS + 16, 16)]
        for v, (is_lo, lo_idx, hi_idx) in enumerate(chunks):
            glo = lax.gather(
                wlo, lo_idx[:, None], dnums, (1,),
                mode=lax.GatherScatterMode.PROMISE_IN_BOUNDS,
            )
            ghi = lax.gather(
                whi, hi_idx[:, None], dnums, (1,),
                mode=lax.GatherScatterMode.PROMISE_IN_BOUNDS,
            )
            out_v[h, pl.ds(v * 16, 16)] = jnp.where(is_lo, glo, ghi)
    pltpu.sync_copy(out_v, vb_hbm.at[:, pl.ds(base, _CHUNK)])


def _stream_kernel(vb_ref, out_ref, m_ref, sem_ref):
    h = pl.program_id(0)
    last = pl.num_programs(0) - 1
    buf = h % 2

    def strip_copies(b, hh):
        return [
            pltpu.make_async_copy(
                m_ref.at[b, :, pl.ds(_S * (_NSTRIP - 1 - i), _KL)],
                out_ref.at[0, hh, pl.ds(_S * i, _S), :],
                sem_ref.at[b],
            )
            for i in range(_NSTRIP)
        ]

    @pl.when(h >= 2)
    def _wait_prev():
        for c in strip_copies(buf, h - 2):
            c.wait()

    m_ref[buf] = pltpu.roll(
        jnp.broadcast_to(vb_ref[...].reshape(1, _MW), (_S, _MW)),
        _MW - (_S - 1),
        axis=1,
        stride=1,
        stride_axis=0,
    )

    for c in strip_copies(buf, h):
        c.start()

    @pl.when(h == last)
    def _drain():
        for c in strip_copies(buf, h):
            c.wait()
        for c in strip_copies(1 - buf, h - 1):
            c.wait()


@functools.partial(
    pl.kernel,
    out_type=jax.ShapeDtypeStruct((_NUM_HEADS, _MW), jnp.float32),
    mesh=plsc.VectorSubcoreMesh(core_axis_name="c", subcore_axis_name="s"),
    scratch_types=[
        pltpu.VMEM((_NUM_BUCKETS * _NUM_HEADS,), jnp.float32),
        pltpu.VMEM((16,), jnp.int32),
        pltpu.VMEM((_NUM_HEADS, _CHUNK), jnp.float32),
    ],
)
def _sc_lookup(w_hbm, delta_hbm, vb_hbm, w_v, delta_v, out_v):
    _sc_lookup_kernel(w_hbm, delta_hbm, vb_hbm, w_v, delta_v, out_v)


def kernel(W, q_len, k_len):
    delta = (
        jnp.asarray(k_len, jnp.int32) - _KL
        - (jnp.asarray(q_len, jnp.int32) - _QL)
    )
    vbase = _sc_lookup(
        W.T.reshape(_NUM_HEADS * _NUM_BUCKETS),
        jnp.full((16,), 1, jnp.int32) * delta,
    )
    vbase = vbase.reshape(_NUM_HEADS, 1, _MW)
    out = pl.pallas_call(
        _stream_kernel,
        grid=(_NUM_HEADS,),
        in_specs=[pl.BlockSpec((1, 1, _MW), lambda h: (h, 0, 0))],
        out_specs=pl.BlockSpec(memory_space=pl.ANY),
        out_shape=jax.ShapeDtypeStruct((1, _NUM_HEADS, _QL, _KL), jnp.float32),
        scratch_shapes=[
            pltpu.VMEM((2, _S, _MW), jnp.float32),
            pltpu.SemaphoreType.DMA((2,)),
        ],
    )(vbase)
    return out
